# baseline (speedup 1.0000x reference)
import jax
import jax.numpy as jnp
from jax import lax
from jax.experimental import pallas as pl
from jax.experimental.pallas import tpu as pltpu
from jax.experimental.pallas import tpu_sc as plsc

_MAX_SIZE = 50
_HALF = 128
_BATCH = 64
_ROWS = _MAX_SIZE * _MAX_SIZE
_NW = 32
_NT = 79


def _sc_body(row_hbm, col_hbm, out_hbm, tabv, units, sem):
    c = lax.axis_index("c")
    s = lax.axis_index("s")
    wid = s * 2 + c

    pltpu.sync_copy(row_hbm, tabv.at[pl.ds(0, _MAX_SIZE * _HALF)])
    pltpu.sync_copy(
        col_hbm, tabv.at[pl.ds(_MAX_SIZE * _HALF, _MAX_SIZE * _HALF)]
    )

    def build(u, carry):
        for k in range(8):
            v = tabv[pl.ds(u * _HALF + k * 16, 16)]
            for sl in range(8):
                units[u, sl, pl.ds(k * 16, 16)] = v
        return carry

    lax.fori_loop(0, 2 * _MAX_SIZE, build, 0)

    def fire(t):
        r = jnp.minimum(wid + _NW * t, _ROWS - 1)
        i = r // _MAX_SIZE
        j = r - i * _MAX_SIZE
        for st in range(8):
            pltpu.make_async_copy(
                units.at[i], out_hbm.at[r, st, 0], sem
            ).start()
            pltpu.make_async_copy(
                units.at[_MAX_SIZE + j], out_hbm.at[r, st, 1], sem
            ).start()

    def drain():
        for st in range(8):
            pltpu.make_async_copy(
                units.at[0], out_hbm.at[0, st, 0], sem
            ).wait()
            pltpu.make_async_copy(
                units.at[0], out_hbm.at[0, st, 1], sem
            ).wait()

    def body(t, carry):
        @pl.when(t >= 8)
        def _():
            drain()

        fire(t)
        return carry

    lax.fori_loop(0, _NT, body, 0)
    drain()
    drain()
    drain()
    drain()
    drain()
    drain()
    drain()
    drain()


def sc_kernel(batch_size, row_embed, col_embed):
    zero = (jnp.asarray(batch_size) - _BATCH).astype(row_embed.dtype)
    row_flat = (row_embed + zero).reshape(-1)
    col_flat = (col_embed + zero).reshape(-1)

    mesh = plsc.VectorSubcoreMesh(core_axis_name="c", subcore_axis_name="s")
    run = pl.kernel(
        _sc_body,
        out_type=jax.ShapeDtypeStruct((_ROWS, 8, 2, 8, _HALF), jnp.float32),
        mesh=mesh,
        scratch_types=[
            pltpu.VMEM((2 * _MAX_SIZE * _HALF,), jnp.float32),
            pltpu.VMEM((2 * _MAX_SIZE, 8, _HALF), jnp.float32),
            pltpu.SemaphoreType.DMA,
        ],
    )
    out5 = run(row_flat, col_flat)
    return (
        out5.transpose(1, 3, 0, 2, 4).reshape(_BATCH, _ROWS, 2 * _HALF)
    )


kernel = sc_kernel


# speedup vs baseline: 1.4889x; 1.4889x over previous
import jax
import jax.numpy as jnp
from jax.experimental import pallas as pl
from jax.experimental.pallas import tpu as pltpu

_MAX_SIZE = 50
_HALF = 128
_BATCH = 64
_IPB = 5
_RPB = _IPB * _MAX_SIZE


def _pos_broadcast_kernel(row_ref, col_ref, out_ref):
    g = pl.program_id(0)
    col = col_ref[...]
    colb = jnp.broadcast_to(col[:, None, :], (_MAX_SIZE, _BATCH, _HALF))
    for k in range(_IPB):
        row = row_ref[pl.ds(g * _IPB + k, 1), :]
        out_ref[pl.ds(k * _MAX_SIZE, _MAX_SIZE), :, :_HALF] = (
            jnp.broadcast_to(row[:, None, :], (_MAX_SIZE, _BATCH, _HALF))
        )
        out_ref[pl.ds(k * _MAX_SIZE, _MAX_SIZE), :, _HALF:] = colb


def kernel(batch_size, row_embed, col_embed):
    zero = (jnp.asarray(batch_size) - _BATCH).astype(row_embed.dtype)
    row_embed = row_embed + zero
    col_embed = col_embed + zero

    out = pl.pallas_call(
        _pos_broadcast_kernel,
        grid=(_MAX_SIZE // _IPB,),
        in_specs=[
            pl.BlockSpec((_MAX_SIZE, _HALF), lambda g: (0, 0)),
            pl.BlockSpec((_MAX_SIZE, _HALF), lambda g: (0, 0)),
        ],
        out_specs=pl.BlockSpec(
            (_RPB, _BATCH, 2 * _HALF), lambda g: (g, 0, 0)
        ),
        out_shape=jax.ShapeDtypeStruct(
            (_MAX_SIZE * _MAX_SIZE, _BATCH, 2 * _HALF), row_embed.dtype
        ),
    )(row_embed, col_embed)
    return jnp.transpose(out, (1, 0, 2))
